# Initial kernel scaffold; baseline (speedup 1.0000x reference)
import jax
import jax.numpy as jnp
from jax.experimental import pallas as pl


def kernel(x, node_opcode, dim_feat, layout_feat, tile_feat, batch, edge_index, opcode_embed, W_pre, b_pre, Wl0, bl0, Wr0, rWl0, rbl0, rWr0, Wl1, bl1, Wr1, rWl1, rbl1, rWr1, W_head, b_head):
    raise NotImplementedError("write your pallas kernel here")


# speedup vs baseline: 122.3481x; 122.3481x over previous
import functools

import jax
import jax.numpy as jnp
from jax import lax
from jax.experimental import pallas as pl
from jax.experimental.pallas import tpu as pltpu
from jax.experimental.pallas import tpu_sc as plsc

_N = 50000
_E = 800000
_S = 2
_G = 4
_MID = 32
_H = 16

_BLK = 2000
_CH = 80
_EPT = _E // 16
_NCH = _EPT // _CH
_ST0 = 3128
_STL = _N - 15 * _ST0
_DW = 8

_f32 = jnp.float32


def _stage_a_body(x_ref, opc_ref, dim_ref, lay_ref, bat_ref,
                  opemb_ref, wx_ref, wop_ref, wdim_ref, wlay_ref,
                  tilmat_ref, wtil_ref, bpre_ref, wstk_ref,
                  pf_ref, pr_ref, rfrr_ref):
    t_op = jnp.dot(opemb_ref[...], wop_ref[...], preferred_element_type=_f32)
    oh_op = (opc_ref[...] == lax.broadcasted_iota(jnp.int32, (_BLK, 121), 1)
             ).astype(_f32)
    base = jnp.dot(x_ref[...], wx_ref[...], preferred_element_type=_f32)
    base = base + jnp.dot(dim_ref[...], wdim_ref[...],
                          preferred_element_type=_f32)
    base = base + jnp.dot(oh_op, t_op, preferred_element_type=_f32)
    base = base + bpre_ref[...]

    t_til = jnp.dot(tilmat_ref[...], wtil_ref[...],
                    preferred_element_type=_f32)
    oh_b = (bat_ref[...] == lax.broadcasted_iota(jnp.int32, (_BLK, _G), 1)
            ).astype(_f32)
    lay = lay_ref[...]
    wstk = wstk_ref[...]
    pf, pr, rf, rr = [], [], [], []
    for s in range(_S):
        h = base + jnp.dot(lay[:, s * 18:(s + 1) * 18], wlay_ref[...],
                           preferred_element_type=_f32)
        h = h + jnp.dot(oh_b, t_til[s * _G:(s + 1) * _G],
                        preferred_element_type=_f32)
        h = jnp.maximum(h, 0.0)
        o = jnp.dot(h, wstk, preferred_element_type=_f32)
        pf.append(o[:, 0:16])
        pr.append(o[:, 16:32])
        rf.append(o[:, 32:48])
        rr.append(o[:, 48:64])
    pf_ref[...] = jnp.concatenate(pf, axis=1)
    pr_ref[...] = jnp.concatenate(pr, axis=1)
    rfrr_ref[...] = jnp.concatenate(rf + rr, axis=1)


def _stage_a(x, opc2, dim2, lay2, bat2, opemb, wx, wop, wdim, wlay,
             tilmat, wtil, bpre, wstk):
    grid = _N // _BLK
    row = lambda i: (i, 0)
    full = lambda i: (0, 0)
    return pl.pallas_call(
        _stage_a_body,
        grid=(grid,),
        in_specs=[
            pl.BlockSpec((_BLK, 53), row),
            pl.BlockSpec((_BLK, 1), row),
            pl.BlockSpec((_BLK, 138), row),
            pl.BlockSpec((_BLK, 36), row),
            pl.BlockSpec((_BLK, 1), row),
            pl.BlockSpec((121, 16), full),
            pl.BlockSpec((53, 32), full),
            pl.BlockSpec((16, 32), full),
            pl.BlockSpec((138, 32), full),
            pl.BlockSpec((18, 32), full),
            pl.BlockSpec((_S * _G, 24), full),
            pl.BlockSpec((24, 32), full),
            pl.BlockSpec((1, 32), full),
            pl.BlockSpec((32, 64), full),
        ],
        out_specs=[
            pl.BlockSpec((_BLK, 32), row),
            pl.BlockSpec((_BLK, 32), row),
            pl.BlockSpec((_BLK, 64), row),
        ],
        out_shape=[
            jax.ShapeDtypeStruct((_N, 32), _f32),
            jax.ShapeDtypeStruct((_N, 32), _f32),
            jax.ShapeDtypeStruct((_N, 64), _f32),
        ],
    )(x, opc2, dim2, lay2, bat2, opemb, wx, wop, wdim, wlay,
      tilmat, wtil, bpre, wstk)


def _combine_h(aggf, aggr, rfrr, invdd, invds, bl, rbl):
    hs = []
    for s in range(_S):
        fwd = aggf[:, s * 16:(s + 1) * 16] * invdd + bl + \
            rfrr[:, s * 16:(s + 1) * 16]
        rev = aggr[:, s * 16:(s + 1) * 16] * invds + rbl + \
            rfrr[:, 32 + s * 16:32 + (s + 1) * 16]
        hs.append(jnp.maximum(jnp.concatenate([fwd, rev], axis=1), 0.0))
    return hs


def _stage_b_body(aggf_ref, aggr_ref, rfrr_ref, dd_ref, ds_ref,
                  bl_ref, rbl_ref, wstk_ref,
                  pf_ref, pr_ref, rfrr_out_ref):
    invdd = 1.0 / jnp.maximum(dd_ref[...][:, 0:1], 1.0)
    invds = 1.0 / jnp.maximum(ds_ref[...][:, 0:1], 1.0)
    hs = _combine_h(aggf_ref[...], aggr_ref[...], rfrr_ref[...],
                    invdd, invds, bl_ref[...], rbl_ref[...])
    wstk = wstk_ref[...]
    pf, pr, rf, rr = [], [], [], []
    for s in range(_S):
        o = jnp.dot(hs[s], wstk, preferred_element_type=_f32)
        pf.append(o[:, 0:16])
        pr.append(o[:, 16:32])
        rf.append(o[:, 32:48])
        rr.append(o[:, 48:64])
    pf_ref[...] = jnp.concatenate(pf, axis=1)
    pr_ref[...] = jnp.concatenate(pr, axis=1)
    rfrr_out_ref[...] = jnp.concatenate(rf + rr, axis=1)


def _stage_b(aggf, aggr, rfrr, deg_d, deg_s, bl, rbl, wstk):
    grid = _N // _BLK
    row = lambda i: (i, 0)
    full = lambda i: (0, 0)
    return pl.pallas_call(
        _stage_b_body,
        grid=(grid,),
        in_specs=[
            pl.BlockSpec((_BLK, 32), row),
            pl.BlockSpec((_BLK, 32), row),
            pl.BlockSpec((_BLK, 64), row),
            pl.BlockSpec((_BLK, _DW), row),
            pl.BlockSpec((_BLK, _DW), row),
            pl.BlockSpec((1, 16), full),
            pl.BlockSpec((1, 16), full),
            pl.BlockSpec((32, 64), full),
        ],
        out_specs=[
            pl.BlockSpec((_BLK, 32), row),
            pl.BlockSpec((_BLK, 32), row),
            pl.BlockSpec((_BLK, 64), row),
        ],
        out_shape=[
            jax.ShapeDtypeStruct((_N, 32), _f32),
            jax.ShapeDtypeStruct((_N, 32), _f32),
            jax.ShapeDtypeStruct((_N, 64), _f32),
        ],
    )(aggf, aggr, rfrr, deg_d, deg_s, bl, rbl, wstk)


def _stage_c_body(aggf_ref, aggr_ref, rfrr_ref, dd_ref, ds_ref,
                  bat_ref, bl_ref, rbl_ref, wh_ref, bh_ref,
                  out_ref, acc_ref):
    i = pl.program_id(0)
    invdd = 1.0 / jnp.maximum(dd_ref[...][:, 0:1], 1.0)
    invds = 1.0 / jnp.maximum(ds_ref[...][:, 0:1], 1.0)
    hs = _combine_h(aggf_ref[...], aggr_ref[...], rfrr_ref[...],
                    invdd, invds, bl_ref[...], rbl_ref[...])
    oh_b = (bat_ref[...] == lax.broadcasted_iota(jnp.int32, (_BLK, _G), 1)
            ).astype(_f32)
    pooled = jnp.concatenate(
        [lax.dot_general(oh_b, hs[s], (((0,), (0,)), ((), ())),
                         preferred_element_type=_f32) for s in range(_S)],
        axis=0)

    @pl.when(i == 0)
    def _():
        acc_ref[...] = jnp.zeros((_S * _G, 32), _f32)

    acc_ref[...] += pooled

    @pl.when(i == pl.num_programs(0) - 1)
    def _():
        val = jnp.dot(acc_ref[...], wh_ref[...],
                      preferred_element_type=_f32) + bh_ref[...]
        out_ref[...] = jnp.broadcast_to(val, (_S * _G, 128))


def _stage_c(aggf, aggr, rfrr, deg_d, deg_s, bat2, bl, rbl, wh, bh):
    grid = _N // _BLK
    row = lambda i: (i, 0)
    full = lambda i: (0, 0)
    return pl.pallas_call(
        _stage_c_body,
        grid=(grid,),
        in_specs=[
            pl.BlockSpec((_BLK, 32), row),
            pl.BlockSpec((_BLK, 32), row),
            pl.BlockSpec((_BLK, 64), row),
            pl.BlockSpec((_BLK, _DW), row),
            pl.BlockSpec((_BLK, _DW), row),
            pl.BlockSpec((_BLK, 1), row),
            pl.BlockSpec((1, 16), full),
            pl.BlockSpec((1, 16), full),
            pl.BlockSpec((32, 1), full),
            pl.BlockSpec((1, 1), full),
        ],
        out_specs=pl.BlockSpec((_S * _G, 128), full),
        out_shape=jax.ShapeDtypeStruct((_S * _G, 128), _f32),
        scratch_shapes=[pltpu.VMEM((_S * _G, 32), _f32)],
    )(aggf, aggr, rfrr, deg_d, deg_s, bat2, bl, rbl, wh, bh)


def _edge_dir(tab_hbm, gidx_hbm, sidx_hbm, out_hbm, acc_sh,
              visx, vidx, rows, z32_hbm, sid, with_deg,
              dacc_sh=None, dout_hbm=None, ones=None, z8_hbm=None):
    r0 = pl.multiple_of(sid * _ST0, 8)

    @pl.when(sid < 15)
    def _():
        pltpu.sync_copy(z32_hbm, acc_sh.at[pl.ds(r0, _ST0)])
        if with_deg:
            pltpu.sync_copy(z8_hbm, dacc_sh.at[pl.ds(r0, _ST0)])

    @pl.when(sid == 15)
    def _():
        pltpu.sync_copy(z32_hbm.at[pl.ds(0, _STL)],
                        acc_sh.at[pl.ds(r0, _STL)])
        if with_deg:
            pltpu.sync_copy(z8_hbm.at[pl.ds(0, _STL)],
                            dacc_sh.at[pl.ds(r0, _STL)])

    plsc.subcore_barrier()

    ebase = sid * _EPT

    def chunk(c, _):
        b = pl.multiple_of(ebase + c * _CH, 8)
        pltpu.sync_copy(gidx_hbm.at[pl.ds(b, _CH)], visx)
        pltpu.sync_copy(sidx_hbm.at[pl.ds(b, _CH)], vidx)
        pltpu.sync_copy(tab_hbm.at[visx], rows)
        pltpu.sync_copy(rows, acc_sh.at[vidx], add=True)
        if with_deg:
            pltpu.sync_copy(ones, dacc_sh.at[vidx], add=True)
        return 0

    lax.fori_loop(0, _NCH, chunk, 0)
    plsc.subcore_barrier()

    @pl.when(sid < 15)
    def _():
        pltpu.sync_copy(acc_sh.at[pl.ds(r0, _ST0)],
                        out_hbm.at[pl.ds(r0, _ST0)])
        if with_deg:
            pltpu.sync_copy(dacc_sh.at[pl.ds(r0, _ST0)],
                            dout_hbm.at[pl.ds(r0, _ST0)])

    @pl.when(sid == 15)
    def _():
        pltpu.sync_copy(acc_sh.at[pl.ds(r0, _STL)],
                        out_hbm.at[pl.ds(r0, _STL)])
        if with_deg:
            pltpu.sync_copy(dacc_sh.at[pl.ds(r0, _STL)],
                            dout_hbm.at[pl.ds(r0, _STL)])


def _edge_kernel_l0(pf, pr, src, dst, z32, z8, ones8):
    mesh = plsc.VectorSubcoreMesh(core_axis_name="c", subcore_axis_name="s")

    def body(pf_hbm, pr_hbm, src_hbm, dst_hbm, z32_hbm, z8_hbm, ones_hbm,
             aggf_hbm, aggr_hbm, dd_hbm, ds_hbm,
             acc_sh, dacc_sh, visx, vidx, rows, ones):
        cid = lax.axis_index("c")
        sid = lax.axis_index("s")
        pltpu.sync_copy(ones_hbm, ones)

        @pl.when(cid == 0)
        def _():
            _edge_dir(pf_hbm, src_hbm, dst_hbm, aggf_hbm, acc_sh,
                      visx, vidx, rows, z32_hbm, sid, True,
                      dacc_sh, dd_hbm, ones, z8_hbm)

        @pl.when(cid == 1)
        def _():
            _edge_dir(pr_hbm, dst_hbm, src_hbm, aggr_hbm, acc_sh,
                      visx, vidx, rows, z32_hbm, sid, True,
                      dacc_sh, ds_hbm, ones, z8_hbm)

    f = pl.kernel(
        body,
        out_type=[
            jax.ShapeDtypeStruct((_N, 32), _f32),
            jax.ShapeDtypeStruct((_N, 32), _f32),
            jax.ShapeDtypeStruct((_N, _DW), _f32),
            jax.ShapeDtypeStruct((_N, _DW), _f32),
        ],
        mesh=mesh,
        compiler_params=pltpu.CompilerParams(use_tc_tiling_on_sc=False),
        scratch_types=[
            pltpu.MemorySpace.VMEM_SHARED((_N, 32), _f32),
            pltpu.MemorySpace.VMEM_SHARED((_N, _DW), _f32),
            pltpu.VMEM((_CH,), jnp.int32),
            pltpu.VMEM((_CH,), jnp.int32),
            pltpu.VMEM((_CH, 32), _f32),
            pltpu.VMEM((_CH, _DW), _f32),
        ],
    )
    return f(pf, pr, src, dst, z32, z8, ones8)


def _edge_kernel_l1(pf, pr, src, dst, z32):
    mesh = plsc.VectorSubcoreMesh(core_axis_name="c", subcore_axis_name="s")

    def body(pf_hbm, pr_hbm, src_hbm, dst_hbm, z32_hbm, aggf_hbm, aggr_hbm,
             acc_sh, visx, vidx, rows):
        cid = lax.axis_index("c")
        sid = lax.axis_index("s")

        @pl.when(cid == 0)
        def _():
            _edge_dir(pf_hbm, src_hbm, dst_hbm, aggf_hbm, acc_sh,
                      visx, vidx, rows, z32_hbm, sid, False)

        @pl.when(cid == 1)
        def _():
            _edge_dir(pr_hbm, dst_hbm, src_hbm, aggr_hbm, acc_sh,
                      visx, vidx, rows, z32_hbm, sid, False)

    f = pl.kernel(
        body,
        out_type=[
            jax.ShapeDtypeStruct((_N, 32), _f32),
            jax.ShapeDtypeStruct((_N, 32), _f32),
        ],
        mesh=mesh,
        compiler_params=pltpu.CompilerParams(use_tc_tiling_on_sc=False),
        scratch_types=[
            pltpu.MemorySpace.VMEM_SHARED((_N, 32), _f32),
            pltpu.VMEM((_CH,), jnp.int32),
            pltpu.VMEM((_CH,), jnp.int32),
            pltpu.VMEM((_CH, 32), _f32),
        ],
    )
    return f(pf, pr, src, dst, z32)


def kernel(x, node_opcode, dim_feat, layout_feat, tile_feat, batch, edge_index,
           opcode_embed, W_pre, b_pre,
           Wl0, bl0, Wr0, rWl0, rbl0, rWr0,
           Wl1, bl1, Wr1, rWl1, rbl1, rWr1,
           W_head, b_head):
    opc2 = node_opcode.astype(jnp.int32).reshape(_N, 1)
    bat2 = batch.astype(jnp.int32).reshape(_N, 1)
    dim2 = dim_feat.reshape(_N, 138)
    lay2 = layout_feat.reshape(_N, 36)
    tilmat = jnp.transpose(tile_feat, (1, 0, 2, 3)).reshape(_S * _G, 24)
    src = edge_index[0].astype(jnp.int32)
    dst = edge_index[1].astype(jnp.int32)

    wx = W_pre[0:53]
    wop = W_pre[53:69]
    wdim = W_pre[69:207]
    wlay = W_pre[207:225]
    wtil = W_pre[225:249]
    bpre = b_pre.reshape(1, 32)
    wstk0 = jnp.concatenate([Wl0, rWl0, Wr0, rWr0], axis=1)
    wstk1 = jnp.concatenate([Wl1, rWl1, Wr1, rWr1], axis=1)
    bl0r = bl0.reshape(1, 16)
    rbl0r = rbl0.reshape(1, 16)
    bl1r = bl1.reshape(1, 16)
    rbl1r = rbl1.reshape(1, 16)
    bh = b_head.reshape(1, 1)

    z32 = jnp.zeros((_ST0, 32), _f32)
    z8 = jnp.zeros((_ST0, _DW), _f32)
    ones8 = jnp.ones((_CH, _DW), _f32)

    pf0, pr0, rfrr0 = _stage_a(x, opc2, dim2, lay2, bat2, opcode_embed,
                               wx, wop, wdim, wlay, tilmat, wtil, bpre, wstk0)
    aggf0, aggr0, deg_d, deg_s = _edge_kernel_l0(pf0, pr0, src, dst,
                                                 z32, z8, ones8)
    pf1, pr1, rfrr1 = _stage_b(aggf0, aggr0, rfrr0, deg_d, deg_s,
                               bl0r, rbl0r, wstk1)
    aggf1, aggr1 = _edge_kernel_l1(pf1, pr1, src, dst, z32)
    res = _stage_c(aggf1, aggr1, rfrr1, deg_d, deg_s, bat2,
                   bl1r, rbl1r, W_head, bh)
    out = res[:, 0:1].reshape(_S, _G, 1)
    return jnp.transpose(out, (1, 0, 2))


# speedup vs baseline: 390.0797x; 3.1883x over previous
import functools

import jax
import jax.numpy as jnp
from jax import lax
from jax.experimental import pallas as pl
from jax.experimental.pallas import tpu as pltpu
from jax.experimental.pallas import tpu_sc as plsc

_N = 50000
_E = 800000
_S = 2
_G = 4
_MID = 32
_H = 16

_BLK = 2000
_CH = 80
_KB = 5
_EB = _CH * _KB
_EPT = _E // 16
_NB = _EPT // _EB
_ST0 = 3128
_STL = _N - 15 * _ST0
_DW = 8

_f32 = jnp.float32


def _stage_a_body(x_ref, opc_ref, dim_ref, lay_ref, bat_ref,
                  opemb_ref, wx_ref, wop_ref, wdim_ref, wlay_ref,
                  tilmat_ref, wtil_ref, bpre_ref, wstk_ref,
                  pf_ref, pr_ref, rfrr_ref):
    t_op = jnp.dot(opemb_ref[...], wop_ref[...], preferred_element_type=_f32)
    oh_op = (opc_ref[...] == lax.broadcasted_iota(jnp.int32, (_BLK, 121), 1)
             ).astype(_f32)
    base = jnp.dot(x_ref[...], wx_ref[...], preferred_element_type=_f32)
    base = base + jnp.dot(dim_ref[...], wdim_ref[...],
                          preferred_element_type=_f32)
    base = base + jnp.dot(oh_op, t_op, preferred_element_type=_f32)
    base = base + bpre_ref[...]

    t_til = jnp.dot(tilmat_ref[...], wtil_ref[...],
                    preferred_element_type=_f32)
    oh_b = (bat_ref[...] == lax.broadcasted_iota(jnp.int32, (_BLK, _G), 1)
            ).astype(_f32)
    lay = lay_ref[...]
    wstk = wstk_ref[...]
    pf, pr, rf, rr = [], [], [], []
    for s in range(_S):
        h = base + jnp.dot(lay[:, s * 18:(s + 1) * 18], wlay_ref[...],
                           preferred_element_type=_f32)
        h = h + jnp.dot(oh_b, t_til[s * _G:(s + 1) * _G],
                        preferred_element_type=_f32)
        h = jnp.maximum(h, 0.0)
        o = jnp.dot(h, wstk, preferred_element_type=_f32)
        pf.append(o[:, 0:16])
        pr.append(o[:, 16:32])
        rf.append(o[:, 32:48])
        rr.append(o[:, 48:64])
    pf_ref[...] = jnp.concatenate(pf, axis=1)
    pr_ref[...] = jnp.concatenate(pr, axis=1)
    rfrr_ref[...] = jnp.concatenate(rf + rr, axis=1)


def _stage_a(x, opc2, dim2, lay2, bat2, opemb, wx, wop, wdim, wlay,
             tilmat, wtil, bpre, wstk):
    grid = _N // _BLK
    row = lambda i: (i, 0)
    full = lambda i: (0, 0)
    return pl.pallas_call(
        _stage_a_body,
        grid=(grid,),
        in_specs=[
            pl.BlockSpec((_BLK, 53), row),
            pl.BlockSpec((_BLK, 1), row),
            pl.BlockSpec((_BLK, 138), row),
            pl.BlockSpec((_BLK, 36), row),
            pl.BlockSpec((_BLK, 1), row),
            pl.BlockSpec((121, 16), full),
            pl.BlockSpec((53, 32), full),
            pl.BlockSpec((16, 32), full),
            pl.BlockSpec((138, 32), full),
            pl.BlockSpec((18, 32), full),
            pl.BlockSpec((_S * _G, 24), full),
            pl.BlockSpec((24, 32), full),
            pl.BlockSpec((1, 32), full),
            pl.BlockSpec((32, 64), full),
        ],
        out_specs=[
            pl.BlockSpec((_BLK, 32), row),
            pl.BlockSpec((_BLK, 32), row),
            pl.BlockSpec((_BLK, 64), row),
        ],
        out_shape=[
            jax.ShapeDtypeStruct((_N, 32), _f32),
            jax.ShapeDtypeStruct((_N, 32), _f32),
            jax.ShapeDtypeStruct((_N, 64), _f32),
        ],
    )(x, opc2, dim2, lay2, bat2, opemb, wx, wop, wdim, wlay,
      tilmat, wtil, bpre, wstk)


def _combine_h(aggf, aggr, rfrr, invdd, invds, bl, rbl):
    hs = []
    for s in range(_S):
        fwd = aggf[:, s * 16:(s + 1) * 16] * invdd + bl + \
            rfrr[:, s * 16:(s + 1) * 16]
        rev = aggr[:, s * 16:(s + 1) * 16] * invds + rbl + \
            rfrr[:, 32 + s * 16:32 + (s + 1) * 16]
        hs.append(jnp.maximum(jnp.concatenate([fwd, rev], axis=1), 0.0))
    return hs


def _stage_b_body(aggf_ref, aggr_ref, rfrr_ref, dd_ref, ds_ref,
                  bl_ref, rbl_ref, wstk_ref,
                  pf_ref, pr_ref, rfrr_out_ref):
    invdd = 1.0 / jnp.maximum(dd_ref[...][:, 0:1], 1.0)
    invds = 1.0 / jnp.maximum(ds_ref[...][:, 0:1], 1.0)
    hs = _combine_h(aggf_ref[...], aggr_ref[...], rfrr_ref[...],
                    invdd, invds, bl_ref[...], rbl_ref[...])
    wstk = wstk_ref[...]
    pf, pr, rf, rr = [], [], [], []
    for s in range(_S):
        o = jnp.dot(hs[s], wstk, preferred_element_type=_f32)
        pf.append(o[:, 0:16])
        pr.append(o[:, 16:32])
        rf.append(o[:, 32:48])
        rr.append(o[:, 48:64])
    pf_ref[...] = jnp.concatenate(pf, axis=1)
    pr_ref[...] = jnp.concatenate(pr, axis=1)
    rfrr_out_ref[...] = jnp.concatenate(rf + rr, axis=1)


def _stage_b(aggf, aggr, rfrr, deg_d, deg_s, bl, rbl, wstk):
    grid = _N // _BLK
    row = lambda i: (i, 0)
    full = lambda i: (0, 0)
    return pl.pallas_call(
        _stage_b_body,
        grid=(grid,),
        in_specs=[
            pl.BlockSpec((_BLK, 32), row),
            pl.BlockSpec((_BLK, 32), row),
            pl.BlockSpec((_BLK, 64), row),
            pl.BlockSpec((_BLK, _DW), row),
            pl.BlockSpec((_BLK, _DW), row),
            pl.BlockSpec((1, 16), full),
            pl.BlockSpec((1, 16), full),
            pl.BlockSpec((32, 64), full),
        ],
        out_specs=[
            pl.BlockSpec((_BLK, 32), row),
            pl.BlockSpec((_BLK, 32), row),
            pl.BlockSpec((_BLK, 64), row),
        ],
        out_shape=[
            jax.ShapeDtypeStruct((_N, 32), _f32),
            jax.ShapeDtypeStruct((_N, 32), _f32),
            jax.ShapeDtypeStruct((_N, 64), _f32),
        ],
    )(aggf, aggr, rfrr, deg_d, deg_s, bl, rbl, wstk)


def _stage_c_body(aggf_ref, aggr_ref, rfrr_ref, dd_ref, ds_ref,
                  bat_ref, bl_ref, rbl_ref, wh_ref, bh_ref,
                  out_ref, acc_ref):
    i = pl.program_id(0)
    invdd = 1.0 / jnp.maximum(dd_ref[...][:, 0:1], 1.0)
    invds = 1.0 / jnp.maximum(ds_ref[...][:, 0:1], 1.0)
    hs = _combine_h(aggf_ref[...], aggr_ref[...], rfrr_ref[...],
                    invdd, invds, bl_ref[...], rbl_ref[...])
    oh_b = (bat_ref[...] == lax.broadcasted_iota(jnp.int32, (_BLK, _G), 1)
            ).astype(_f32)
    pooled = jnp.concatenate(
        [lax.dot_general(oh_b, hs[s], (((0,), (0,)), ((), ())),
                         preferred_element_type=_f32) for s in range(_S)],
        axis=0)

    @pl.when(i == 0)
    def _():
        acc_ref[...] = jnp.zeros((_S * _G, 32), _f32)

    acc_ref[...] += pooled

    @pl.when(i == pl.num_programs(0) - 1)
    def _():
        val = jnp.dot(acc_ref[...], wh_ref[...],
                      preferred_element_type=_f32) + bh_ref[...]
        out_ref[...] = jnp.broadcast_to(val, (_S * _G, 128))


def _stage_c(aggf, aggr, rfrr, deg_d, deg_s, bat2, bl, rbl, wh, bh):
    grid = _N // _BLK
    row = lambda i: (i, 0)
    full = lambda i: (0, 0)
    return pl.pallas_call(
        _stage_c_body,
        grid=(grid,),
        in_specs=[
            pl.BlockSpec((_BLK, 32), row),
            pl.BlockSpec((_BLK, 32), row),
            pl.BlockSpec((_BLK, 64), row),
            pl.BlockSpec((_BLK, _DW), row),
            pl.BlockSpec((_BLK, _DW), row),
            pl.BlockSpec((_BLK, 1), row),
            pl.BlockSpec((1, 16), full),
            pl.BlockSpec((1, 16), full),
            pl.BlockSpec((32, 1), full),
            pl.BlockSpec((1, 1), full),
        ],
        out_specs=pl.BlockSpec((_S * _G, 128), full),
        out_shape=jax.ShapeDtypeStruct((_S * _G, 128), _f32),
        scratch_shapes=[pltpu.VMEM((_S * _G, 32), _f32)],
    )(aggf, aggr, rfrr, deg_d, deg_s, bat2, bl, rbl, wh, bh)


def _edge_dir(tab_hbm, gidx_hbm, sidx_hbm, out_hbm, acc_sh,
              gix, six, rows, sem_idx, sem_g, sem_s,
              z32_hbm, sid):
    r0 = pl.multiple_of(sid * _ST0, 8)

    @pl.when(sid < 15)
    def _():
        pltpu.sync_copy(z32_hbm, acc_sh.at[pl.ds(r0, _ST0)])

    @pl.when(sid == 15)
    def _():
        pltpu.sync_copy(z32_hbm.at[pl.ds(0, _STL)],
                        acc_sh.at[pl.ds(r0, _STL)])

    plsc.subcore_barrier()

    rb0 = sid * _NB

    def issue_idx(blk):
        ib = blk % 3
        pltpu.async_copy(gidx_hbm.at[rb0 + blk], gix.at[ib], sem_idx.at[ib])
        pltpu.async_copy(sidx_hbm.at[rb0 + blk], six.at[ib], sem_idx.at[ib])

    def wait_idx(blk):
        ib = blk % 3
        pltpu.make_async_copy(gidx_hbm.at[rb0 + blk], gix.at[ib],
                              sem_idx.at[ib]).wait()
        pltpu.make_async_copy(sidx_hbm.at[rb0 + blk], six.at[ib],
                              sem_idx.at[ib]).wait()

    def issue_gathers(blk):
        ib, p = blk % 3, blk % 2
        for j in range(_KB):
            pltpu.async_copy(tab_hbm.at[gix.at[ib, j]],
                             rows.at[p, j], sem_g.at[p])

    def wait_gathers(blk):
        ib, p = blk % 3, blk % 2
        for j in range(_KB):
            pltpu.make_async_copy(tab_hbm.at[gix.at[ib, j]],
                                  rows.at[p, j], sem_g.at[p]).wait()

    def issue_scatters(blk):
        ib, p = blk % 3, blk % 2
        for j in range(_KB):
            pltpu.async_copy(rows.at[p, j], acc_sh.at[six.at[ib, j]],
                             sem_s.at[p], add=True)

    def wait_scatters(blk):
        ib, p = blk % 3, blk % 2
        for j in range(_KB):
            pltpu.make_async_copy(rows.at[p, j], acc_sh.at[six.at[ib, j]],
                                  sem_s.at[p]).wait()

    issue_idx(0)
    issue_idx(1)
    wait_idx(0)
    issue_gathers(0)

    def block_body(b, _):
        @pl.when(b > 0)
        def _():
            wait_scatters(b - 1)

        @pl.when(b + 1 < _NB)
        def _():
            wait_idx(b + 1)
            issue_gathers(b + 1)

        wait_gathers(b)
        issue_scatters(b)

        @pl.when(b + 2 < _NB)
        def _():
            issue_idx(b + 2)

        return 0

    lax.fori_loop(0, _NB, block_body, 0)
    wait_scatters(_NB - 1)
    plsc.subcore_barrier()

    @pl.when(sid < 15)
    def _():
        pltpu.sync_copy(acc_sh.at[pl.ds(r0, _ST0)],
                        out_hbm.at[pl.ds(r0, _ST0)])

    @pl.when(sid == 15)
    def _():
        pltpu.sync_copy(acc_sh.at[pl.ds(r0, _STL)],
                        out_hbm.at[pl.ds(r0, _STL)])


def _edge_kernel(pf, pr, srcg, srcs, dstg, dsts, z32):
    mesh = plsc.VectorSubcoreMesh(core_axis_name="c", subcore_axis_name="s")

    def body(pf_hbm, pr_hbm, srcg_hbm, srcs_hbm, dstg_hbm, dsts_hbm,
             z32_hbm, aggf_hbm, aggr_hbm,
             acc_sh, gix, six, rows, sem_idx, sem_g, sem_s):
        cid = lax.axis_index("c")
        sid = lax.axis_index("s")

        @pl.when(cid == 0)
        def _():
            _edge_dir(pf_hbm, srcg_hbm, dsts_hbm, aggf_hbm, acc_sh,
                      gix, six, rows, sem_idx, sem_g, sem_s, z32_hbm, sid)

        @pl.when(cid == 1)
        def _():
            _edge_dir(pr_hbm, dstg_hbm, srcs_hbm, aggr_hbm, acc_sh,
                      gix, six, rows, sem_idx, sem_g, sem_s, z32_hbm, sid)

    f = pl.kernel(
        body,
        out_type=[
            jax.ShapeDtypeStruct((_N, 32), _f32),
            jax.ShapeDtypeStruct((_N, 32), _f32),
        ],
        mesh=mesh,
        compiler_params=pltpu.CompilerParams(use_tc_tiling_on_sc=False),
        scratch_types=[
            pltpu.MemorySpace.VMEM_SHARED((_N, 32), _f32),
            pltpu.VMEM((3, _KB, _CH), jnp.int32),
            pltpu.VMEM((3, _KB, _CH), jnp.int32),
            pltpu.VMEM((2, _KB, _CH, 32), _f32),
            pltpu.SemaphoreType.DMA((3,)),
            pltpu.SemaphoreType.DMA((2,)),
            pltpu.SemaphoreType.DMA((2,)),
        ],
    )
    return f(pf, pr, srcg, srcs, dstg, dsts, z32)


def _deg_kernel(srcs, dsts, z8, ones8):
    mesh = plsc.VectorSubcoreMesh(core_axis_name="c", subcore_axis_name="s")

    def one_dir(sidx_hbm, dout_hbm, dacc_sh, six, ones, sem_idx, sem_s,
                z8_hbm, sid):
        r0 = pl.multiple_of(sid * _ST0, 8)

        @pl.when(sid < 15)
        def _():
            pltpu.sync_copy(z8_hbm, dacc_sh.at[pl.ds(r0, _ST0)])

        @pl.when(sid == 15)
        def _():
            pltpu.sync_copy(z8_hbm.at[pl.ds(0, _STL)],
                            dacc_sh.at[pl.ds(r0, _STL)])

        plsc.subcore_barrier()
        rb0 = sid * _NB

        def issue_idx(blk):
            ib = blk % 3
            pltpu.async_copy(sidx_hbm.at[rb0 + blk], six.at[ib],
                             sem_idx.at[ib])

        def wait_idx(blk):
            ib = blk % 3
            pltpu.make_async_copy(sidx_hbm.at[rb0 + blk], six.at[ib],
                                  sem_idx.at[ib]).wait()

        def issue_adds(blk):
            ib, p = blk % 3, blk % 2
            for j in range(_KB):
                pltpu.async_copy(ones, dacc_sh.at[six.at[ib, j]],
                                 sem_s.at[p], add=True)

        def wait_adds(blk):
            ib, p = blk % 3, blk % 2
            for j in range(_KB):
                pltpu.make_async_copy(ones, dacc_sh.at[six.at[ib, j]],
                                      sem_s.at[p]).wait()

        issue_idx(0)
        issue_idx(1)

        def block_body(b, _):
            @pl.when(b > 0)
            def _():
                wait_adds(b - 1)

            wait_idx(b)
            issue_adds(b)

            @pl.when(b + 2 < _NB)
            def _():
                issue_idx(b + 2)

            return 0

        lax.fori_loop(0, _NB, block_body, 0)
        wait_adds(_NB - 1)
        plsc.subcore_barrier()

        @pl.when(sid < 15)
        def _():
            pltpu.sync_copy(dacc_sh.at[pl.ds(r0, _ST0)],
                            dout_hbm.at[pl.ds(r0, _ST0)])

        @pl.when(sid == 15)
        def _():
            pltpu.sync_copy(dacc_sh.at[pl.ds(r0, _STL)],
                            dout_hbm.at[pl.ds(r0, _STL)])

    def body(srcs_hbm, dsts_hbm, z8_hbm, ones_hbm, dd_hbm, ds_hbm,
             dacc_sh, six, ones, sem_idx, sem_s):
        cid = lax.axis_index("c")
        sid = lax.axis_index("s")
        pltpu.sync_copy(ones_hbm, ones)

        @pl.when(cid == 0)
        def _():
            one_dir(dsts_hbm, dd_hbm, dacc_sh, six, ones, sem_idx, sem_s,
                    z8_hbm, sid)

        @pl.when(cid == 1)
        def _():
            one_dir(srcs_hbm, ds_hbm, dacc_sh, six, ones, sem_idx, sem_s,
                    z8_hbm, sid)

    f = pl.kernel(
        body,
        out_type=[
            jax.ShapeDtypeStruct((_N, _DW), _f32),
            jax.ShapeDtypeStruct((_N, _DW), _f32),
        ],
        mesh=mesh,
        compiler_params=pltpu.CompilerParams(use_tc_tiling_on_sc=False),
        scratch_types=[
            pltpu.MemorySpace.VMEM_SHARED((_N, _DW), _f32),
            pltpu.VMEM((3, _KB, _CH), jnp.int32),
            pltpu.VMEM((_CH, _DW), _f32),
            pltpu.SemaphoreType.DMA((3,)),
            pltpu.SemaphoreType.DMA((2,)),
        ],
    )
    return f(srcs, dsts, z8, ones8)


def kernel(x, node_opcode, dim_feat, layout_feat, tile_feat, batch, edge_index,
           opcode_embed, W_pre, b_pre,
           Wl0, bl0, Wr0, rWl0, rbl0, rWr0,
           Wl1, bl1, Wr1, rWl1, rbl1, rWr1,
           W_head, b_head):
    opc2 = node_opcode.astype(jnp.int32).reshape(_N, 1)
    bat2 = batch.astype(jnp.int32).reshape(_N, 1)
    dim2 = dim_feat.reshape(_N, 138)
    lay2 = layout_feat.reshape(_N, 36)
    tilmat = jnp.transpose(tile_feat, (1, 0, 2, 3)).reshape(_S * _G, 24)
    src = edge_index[0].astype(jnp.int32)
    dst = edge_index[1].astype(jnp.int32)
    nbl = _E // _EB
    srcg = srcs = src.reshape(nbl, _KB, _CH)
    dstg = dsts = dst.reshape(nbl, _KB, _CH)

    wx = W_pre[0:53]
    wop = W_pre[53:69]
    wdim = W_pre[69:207]
    wlay = W_pre[207:225]
    wtil = W_pre[225:249]
    bpre = b_pre.reshape(1, 32)
    wstk0 = jnp.concatenate([Wl0, rWl0, Wr0, rWr0], axis=1)
    wstk1 = jnp.concatenate([Wl1, rWl1, Wr1, rWr1], axis=1)
    bl0r = bl0.reshape(1, 16)
    rbl0r = rbl0.reshape(1, 16)
    bl1r = bl1.reshape(1, 16)
    rbl1r = rbl1.reshape(1, 16)
    bh = b_head.reshape(1, 1)

    z32 = jnp.zeros((_ST0, 32), _f32)
    z8 = jnp.zeros((_ST0, _DW), _f32)
    ones8 = jnp.ones((_CH, _DW), _f32)

    pf0, pr0, rfrr0 = _stage_a(x, opc2, dim2, lay2, bat2, opcode_embed,
                               wx, wop, wdim, wlay, tilmat, wtil, bpre, wstk0)
    deg_d, deg_s = _deg_kernel(srcs, dsts, z8, ones8)
    aggf0, aggr0 = _edge_kernel(pf0, pr0, srcg, srcs, dstg, dsts, z32)
    pf1, pr1, rfrr1 = _stage_b(aggf0, aggr0, rfrr0, deg_d, deg_s,
                               bl0r, rbl0r, wstk1)
    aggf1, aggr1 = _edge_kernel(pf1, pr1, srcg, srcs, dstg, dsts, z32)
    res = _stage_c(aggf1, aggr1, rfrr1, deg_d, deg_s, bat2,
                   bl1r, rbl1r, W_head, bh)
    out = res[:, 0:1].reshape(_S, _G, 1)
    return jnp.transpose(out, (1, 0, 2))
